# Initial kernel scaffold; baseline (speedup 1.0000x reference)
import jax
import jax.numpy as jnp
from jax.experimental import pallas as pl


def kernel(input, target):
    raise NotImplementedError("write your pallas kernel here")


# speedup vs baseline: 4.7615x; 4.7615x over previous
import functools

import jax
import jax.numpy as jnp
import numpy as np
from jax import lax
from jax.experimental import pallas as pl
from jax.experimental.pallas import tpu as pltpu
from jax.experimental.pallas import tpu_sc as plsc

N, V, K = 1024, 100000, 100
CW = 128
NC, NS, L = 2, 16, 16
NW = NC * NS
RPW = N // NW
CHUNK = 20000
NCHUNK = V // CHUNK
SEGV = 50
NSEG = CHUNK // (SEGV * L)
CAP = 2048
TRIGGER = 1024
INT_MIN = np.int32(-2147483648)
NEG_INF = np.float32(-np.inf)


def _iota16():
    return lax.iota(jnp.int32, L)


def _splat_f(x):
    return jnp.full((L,), x, jnp.float32)


def _splat_i(x):
    return jnp.full((L,), x, jnp.int32)


def _scalar(v):
    return jnp.max(v)


def _keymap(v):
    b = plsc.bitcast(v, jnp.int32)
    return jnp.where(b >= 0, b, jnp.bitwise_xor(jnp.bitwise_not(b), INT_MIN))


def _inv_keymap_splat(t):
    ts = _splat_i(t)
    bits = jnp.where(ts >= 0, ts, jnp.bitwise_not(jnp.bitwise_xor(ts, INT_MIN)))
    return plsc.bitcast(bits, jnp.float32)


def _count_ge(keybuf, nv, cand):
    cs = _splat_i(cand)

    def body(j, acc):
        kv = keybuf[pl.ds(j * L, L)]
        return acc + plsc.all_reduce_population_count(kv >= cs)

    acc = lax.fori_loop(0, nv, body, _splat_i(0))
    return _scalar(acc)


def _find_kth_key(candbuf, keybuf, pos_s):
    nv = (pos_s + (L - 1)) // L
    io = _iota16()

    def fill(j, _):
        v = candbuf[pl.ds(j * L, L)]
        k = _keymap(v)
        valid = (j * L + io) < pos_s
        keybuf[pl.ds(j * L, L)] = jnp.where(valid, k, INT_MIN)
        return 0

    lax.fori_loop(0, nv, fill, 0)

    c0 = _count_ge(keybuf, nv, jnp.int32(0))
    t = jnp.where(c0 >= K, jnp.int32(0), INT_MIN)

    def probe(i, t):
        bit = jnp.int32(30) - i
        cand = jnp.bitwise_or(t, lax.shift_left(jnp.int32(1), bit))
        c = _count_ge(keybuf, nv, cand)
        return jnp.where(c >= K, cand, t)

    return lax.fori_loop(0, 31, probe, t)


def _compact_gt(src_ref, dst_ref, pos_s, thr_f):
    io = _iota16()
    nv = (pos_s + (L - 1)) // L

    def body(j, pos):
        v = src_ref[pl.ds(j * L, L)]
        m = (v > thr_f) & ((j * L + io) < pos_s)
        pref = plsc.cumsum(m.astype(jnp.int32))
        idx = pos + pref - 1
        plsc.store_scatter(dst_ref, [idx], v, mask=m)
        return pos + plsc.all_reduce_population_count(m)

    pos = lax.fori_loop(0, nv, body, _splat_i(0))
    return _scalar(pos)


def _sc_body(input_hbm, target_hbm, cand_hbm, zt_hbm,
             databuf, candbuf, keybuf, outstage, tgtstage, ztstage, dsem):
    wid = lax.axis_index("s") * NC + lax.axis_index("c")
    base = wid * RPW
    io = _iota16()

    pltpu.sync_copy(target_hbm.at[pl.ds(base, RPW)], tgtstage)

    def do_reselect(pos_s):
        t = _find_kth_key(candbuf, keybuf, pos_s)
        thr_f = _inv_keymap_splat(t)
        c = _compact_gt(candbuf, candbuf, pos_s, thr_f)
        for m in range(7):
            idx = c + m * L + io
            plsc.store_scatter(candbuf, [idx], thr_f, mask=idx < K)
        return thr_f, _splat_i(K)

    def row_body(rl, zt_carry):
        row = base + rl
        tg = tgtstage[pl.ds((rl // L) * L, L)]
        t_r = jnp.sum(jnp.where(io == (rl % L), tg, 0))
        tchunk = t_r // CHUNK

        def chunk_body(c, carry):
            thr_f, pos, ztv = carry
            pltpu.sync_copy(input_hbm.at[row, pl.ds(c * CHUNK, CHUNK)],
                            databuf)

            def seg_body(s, carry2):
                thr_f, pos = carry2

                def vec_body(i, pos):
                    v = databuf[pl.ds(i * L, L)]
                    m = v > thr_f
                    pref = plsc.cumsum(m.astype(jnp.int32))
                    idx = pos + pref - 1
                    plsc.store_scatter(candbuf, [idx], v, mask=m)
                    return pos + plsc.all_reduce_population_count(m)

                pos = lax.fori_loop(s * SEGV, (s + 1) * SEGV, vec_body, pos)
                pos_s = _scalar(pos)
                thr_f, pos = lax.cond(pos_s > TRIGGER,
                                      lambda: do_reselect(pos_s),
                                      lambda: (thr_f, pos))
                return thr_f, pos

            thr_f, pos = lax.fori_loop(0, NSEG, seg_body, (thr_f, pos))

            o = jnp.clip(t_r - c * CHUNK, 0, CHUNK - 1)
            al = (o // L) * L
            v16 = databuf[pl.ds(al, L)]
            val = jnp.sum(jnp.where(io == (o - al), v16, jnp.float32(0.0)))
            hit = (tchunk == c).astype(jnp.float32)
            ztv = ztv + hit * val * jnp.where(io == (rl % L), 1.0, 0.0)
            return thr_f, pos, ztv

        thr_f, pos, zt_carry = lax.fori_loop(
            0, NCHUNK, chunk_body,
            (_splat_f(NEG_INF), _splat_i(0), zt_carry))

        pos_s = _scalar(pos)
        t100 = _find_kth_key(candbuf, keybuf, pos_s)
        tf = _inv_keymap_splat(t100)
        for m in range(CW // L):
            outstage[pl.ds(m * L, L)] = jnp.where(m * L + io < K, tf, NEG_INF)
        cnt = _compact_gt(candbuf, outstage, pos_s, tf)
        del cnt
        pltpu.sync_copy(outstage, cand_hbm.at[row])
        return zt_carry

    zt = lax.fori_loop(0, L, row_body, _splat_f(0.0))
    ztstage[pl.ds(0, L)] = zt
    zt2 = lax.fori_loop(L, RPW, row_body, _splat_f(0.0))
    ztstage[pl.ds(L, L)] = zt2
    pltpu.sync_copy(ztstage, zt_hbm.at[pl.ds(base, RPW)])


@functools.partial(jax.jit, static_argnums=())
def _sc_topk(input, target):
    mesh = plsc.VectorSubcoreMesh(core_axis_name="c", subcore_axis_name="s",
                                  num_cores=NC, num_subcores=NS)
    f = pl.kernel(
        _sc_body,
        out_type=(
            jax.ShapeDtypeStruct((N, CW), jnp.float32),
            jax.ShapeDtypeStruct((N,), jnp.float32),
        ),
        mesh=mesh,
        scratch_types=[
            pltpu.VMEM((CHUNK,), jnp.float32),
            pltpu.VMEM((CAP,), jnp.float32),
            pltpu.VMEM((CAP,), jnp.int32),
            pltpu.VMEM((CW,), jnp.float32),
            pltpu.VMEM((RPW,), jnp.int32),
            pltpu.VMEM((RPW,), jnp.float32),
            pltpu.SemaphoreType.DMA,
        ],
        compiler_params=pltpu.CompilerParams(use_tc_tiling_on_sc=False,
                                             needs_layout_passes=False),
    )
    return f(input, target)


def _tail_body(cand_ref, zt_ref, out_ref):
    v = cand_ref[...]
    lanes = lax.broadcasted_iota(jnp.int32, v.shape, 1)

    def rolled(x, s):
        left = jnp.concatenate([x[:, s:], x[:, :s]], axis=1)
        right = jnp.concatenate([x[:, -s:], x[:, :-s]], axis=1)
        return jnp.where((lanes & s) == 0, left, right)

    k = 2
    while k <= CW:
        j = k // 2
        while j >= 1:
            p = rolled(v, j)
            take_max = ((lanes & k) == 0) == ((lanes & j) == 0)
            v = jnp.where(take_max, jnp.maximum(v, p), jnp.minimum(v, p))
            j //= 2
        k *= 2

    X = v * 0.5
    valid = lanes < K
    Xs = jnp.where(valid, X, 0.0)
    cum1 = Xs
    cum2 = Xs * Xs
    s = 1
    while s < CW:
        def shr(x, sh):
            return jnp.concatenate(
                [jnp.zeros((x.shape[0], sh), x.dtype), x[:, :-sh]], axis=1)
        cum1 = cum1 + shr(cum1, s)
        cum2 = cum2 + shr(cum2, s)
        s *= 2

    rho = (lanes + 1).astype(jnp.float32)
    mean = cum1 / rho
    meansq = cum2 / rho
    ss = rho * (meansq - mean * mean)
    delta = (1.0 - ss) / rho
    tau = mean - jnp.sqrt(jnp.clip(delta, 0.0, None))
    support_mask = (tau <= X) & valid
    support = jnp.sum(support_mask.astype(jnp.int32), axis=1, keepdims=True)
    tau_star = jnp.sum(jnp.where(lanes == support - 1, tau, 0.0), axis=1,
                       keepdims=True)
    p = jnp.where(valid, jnp.square(jnp.clip(X - tau_star, 0.0, None)), 0.0)
    p15 = p * jnp.sqrt(p)
    omega = (1.0 - jnp.sum(p15, axis=1)) / 0.75
    dot = jnp.sum(p * jnp.where(valid, v, 0.0), axis=1)
    loss = omega + dot - zt_ref[:, 0]
    out_ref[...] = (jnp.sum(loss) / float(N)).reshape(1, 1)


def _tail(cand, zt):
    out = pl.pallas_call(
        _tail_body,
        out_shape=jax.ShapeDtypeStruct((1, 1), jnp.float32),
    )(cand, zt.reshape(N, 1))
    return out[0, 0]


def kernel(input, target):
    cand, zt = _sc_topk(input, target)
    return _tail(cand, zt)


# speedup vs baseline: 6.0184x; 1.2640x over previous
import functools

import jax
import jax.numpy as jnp
import numpy as np
from jax import lax
from jax.experimental import pallas as pl
from jax.experimental.pallas import tpu as pltpu
from jax.experimental.pallas import tpu_sc as plsc

N, V, K = 1024, 100000, 100
CW = 128
NC, NS, L = 2, 16, 16
NW = NC * NS
RPW = N // NW
CHUNK = 20000
NCHUNK = V // CHUNK
SEGV = 50
UNROLL = 10
NSEG = CHUNK // (SEGV * L)
CAP = 2048
TRIGGER = 1024
INT_MIN = np.int32(-2147483648)
NEG_INF = np.float32(-np.inf)


def _iota16():
    return lax.iota(jnp.int32, L)


def _splat_f(x):
    return jnp.full((L,), x, jnp.float32)


def _splat_i(x):
    return jnp.full((L,), x, jnp.int32)


def _scalar(v):
    return jnp.max(v)


def _keymap(v):
    b = plsc.bitcast(v, jnp.int32)
    return jnp.where(b >= 0, b, jnp.bitwise_xor(jnp.bitwise_not(b), INT_MIN))


def _inv_keymap_splat(t):
    ts = _splat_i(t)
    bits = jnp.where(ts >= 0, ts, jnp.bitwise_not(jnp.bitwise_xor(ts, INT_MIN)))
    return plsc.bitcast(bits, jnp.float32)


def _count_ge(keybuf, nv4, cand):
    cs = _splat_i(cand)

    def body(j, acc):
        for u in range(4):
            kv = keybuf[pl.ds((j * 4 + u) * L, L)]
            acc = acc + plsc.all_reduce_population_count(kv >= cs)
        return acc

    acc = lax.fori_loop(0, nv4, body, _splat_i(0))
    return _scalar(acc)


def _find_kth_key(candbuf, keybuf, pos_s):
    nv4 = (pos_s + (4 * L - 1)) // (4 * L)
    io = _iota16()

    def fill(j, _):
        for u in range(4):
            v = candbuf[pl.ds((j * 4 + u) * L, L)]
            k = _keymap(v)
            valid = ((j * 4 + u) * L + io) < pos_s
            keybuf[pl.ds((j * 4 + u) * L, L)] = jnp.where(valid, k, INT_MIN)
        return 0

    lax.fori_loop(0, nv4, fill, 0)

    c0 = _count_ge(keybuf, nv4, jnp.int32(0))
    t = jnp.where(c0 >= K, jnp.int32(0), INT_MIN)

    def probe(i, t):
        bit = jnp.int32(30) - i
        cand = jnp.bitwise_or(t, lax.shift_left(jnp.int32(1), bit))
        c = _count_ge(keybuf, nv4, cand)
        return jnp.where(c >= K, cand, t)

    return lax.fori_loop(0, 31, probe, t)


def _compact_gt(src_ref, dst_ref, pos_s, thr_f):
    io = _iota16()
    nv2 = (pos_s + (2 * L - 1)) // (2 * L)

    def body(j, pos):
        for u in range(2):
            v = src_ref[pl.ds((j * 2 + u) * L, L)]
            m = (v > thr_f) & (((j * 2 + u) * L + io) < pos_s)
            pref = plsc.cumsum(m.astype(jnp.int32))
            idx = pos + pref - 1
            plsc.store_scatter(dst_ref, [idx], v, mask=m)
            pos = pos + plsc.all_reduce_population_count(m)
        return pos

    pos = lax.fori_loop(0, nv2, body, _splat_i(0))
    return _scalar(pos)


def _sc_body(input_hbm, target_hbm, cand_hbm, zt_hbm,
             databuf, candbuf, keybuf, outstage, tgtstage, ztstage, dsem):
    wid = lax.axis_index("s") * NC + lax.axis_index("c")
    base = wid * RPW
    io = _iota16()

    pltpu.sync_copy(target_hbm.at[pl.ds(base, RPW)], tgtstage)

    def do_reselect(pos_s):
        t = _find_kth_key(candbuf, keybuf, pos_s)
        thr_f = _inv_keymap_splat(t)
        c = _compact_gt(candbuf, candbuf, pos_s, thr_f)
        for m in range(7):
            idx = c + m * L + io
            plsc.store_scatter(candbuf, [idx], thr_f, mask=idx < K)
        return thr_f, _splat_i(K)

    def row_body(rl, zt_carry):
        row = base + rl
        tg = tgtstage[pl.ds((rl // L) * L, L)]
        t_r = jnp.sum(jnp.where(io == (rl % L), tg, 0))
        tchunk = t_r // CHUNK

        def chunk_body(c, carry):
            thr_f, pos, ztv = carry
            pltpu.sync_copy(input_hbm.at[row, pl.ds(c * CHUNK, CHUNK)],
                            databuf)

            def seg_body(s, carry2):
                thr_f, pos = carry2

                def vec_body(b, pos):
                    i0 = s * SEGV + b * UNROLL
                    for u in range(UNROLL):
                        v = databuf[pl.ds((i0 + u) * L, L)]
                        m = v > thr_f
                        pref = plsc.cumsum(m.astype(jnp.int32))
                        idx = pos + pref - 1
                        plsc.store_scatter(candbuf, [idx], v, mask=m)
                        pos = pos + plsc.all_reduce_population_count(m)
                    return pos

                pos = lax.fori_loop(0, SEGV // UNROLL, vec_body, pos)
                pos_s = _scalar(pos)
                thr_f, pos = lax.cond(pos_s > TRIGGER,
                                      lambda: do_reselect(pos_s),
                                      lambda: (thr_f, pos))
                return thr_f, pos

            thr_f, pos = lax.fori_loop(0, NSEG, seg_body, (thr_f, pos))

            o = jnp.clip(t_r - c * CHUNK, 0, CHUNK - 1)
            al = (o // L) * L
            v16 = databuf[pl.ds(al, L)]
            val = jnp.sum(jnp.where(io == (o - al), v16, jnp.float32(0.0)))
            hit = (tchunk == c).astype(jnp.float32)
            ztv = ztv + hit * val * jnp.where(io == (rl % L), 1.0, 0.0)
            return thr_f, pos, ztv

        thr_f, pos, zt_carry = lax.fori_loop(
            0, NCHUNK, chunk_body,
            (_splat_f(NEG_INF), _splat_i(0), zt_carry))

        pos_s = _scalar(pos)
        t100 = _find_kth_key(candbuf, keybuf, pos_s)
        tf = _inv_keymap_splat(t100)
        for m in range(CW // L):
            outstage[pl.ds(m * L, L)] = jnp.where(m * L + io < K, tf, NEG_INF)
        cnt = _compact_gt(candbuf, outstage, pos_s, tf)
        del cnt
        pltpu.sync_copy(outstage, cand_hbm.at[row])
        return zt_carry

    zt = lax.fori_loop(0, L, row_body, _splat_f(0.0))
    ztstage[pl.ds(0, L)] = zt
    zt2 = lax.fori_loop(L, RPW, row_body, _splat_f(0.0))
    ztstage[pl.ds(L, L)] = zt2
    pltpu.sync_copy(ztstage, zt_hbm.at[pl.ds(base, RPW)])


@functools.partial(jax.jit, static_argnums=())
def _sc_topk(input, target):
    mesh = plsc.VectorSubcoreMesh(core_axis_name="c", subcore_axis_name="s",
                                  num_cores=NC, num_subcores=NS)
    f = pl.kernel(
        _sc_body,
        out_type=(
            jax.ShapeDtypeStruct((N, CW), jnp.float32),
            jax.ShapeDtypeStruct((N,), jnp.float32),
        ),
        mesh=mesh,
        scratch_types=[
            pltpu.VMEM((CHUNK,), jnp.float32),
            pltpu.VMEM((CAP,), jnp.float32),
            pltpu.VMEM((CAP,), jnp.int32),
            pltpu.VMEM((CW,), jnp.float32),
            pltpu.VMEM((RPW,), jnp.int32),
            pltpu.VMEM((RPW,), jnp.float32),
            pltpu.SemaphoreType.DMA,
        ],
        compiler_params=pltpu.CompilerParams(use_tc_tiling_on_sc=False,
                                             needs_layout_passes=False),
    )
    return f(input, target)


def _tail_body(cand_ref, zt_ref, out_ref):
    v = cand_ref[...]
    lanes = lax.broadcasted_iota(jnp.int32, v.shape, 1)

    def rolled(x, s):
        left = jnp.concatenate([x[:, s:], x[:, :s]], axis=1)
        right = jnp.concatenate([x[:, -s:], x[:, :-s]], axis=1)
        return jnp.where((lanes & s) == 0, left, right)

    k = 2
    while k <= CW:
        j = k // 2
        while j >= 1:
            p = rolled(v, j)
            take_max = ((lanes & k) == 0) == ((lanes & j) == 0)
            v = jnp.where(take_max, jnp.maximum(v, p), jnp.minimum(v, p))
            j //= 2
        k *= 2

    X = v * 0.5
    valid = lanes < K
    Xs = jnp.where(valid, X, 0.0)
    cum1 = Xs
    cum2 = Xs * Xs
    s = 1
    while s < CW:
        def shr(x, sh):
            return jnp.concatenate(
                [jnp.zeros((x.shape[0], sh), x.dtype), x[:, :-sh]], axis=1)
        cum1 = cum1 + shr(cum1, s)
        cum2 = cum2 + shr(cum2, s)
        s *= 2

    rho = (lanes + 1).astype(jnp.float32)
    mean = cum1 / rho
    meansq = cum2 / rho
    ss = rho * (meansq - mean * mean)
    delta = (1.0 - ss) / rho
    tau = mean - jnp.sqrt(jnp.clip(delta, 0.0, None))
    support_mask = (tau <= X) & valid
    support = jnp.sum(support_mask.astype(jnp.int32), axis=1, keepdims=True)
    tau_star = jnp.sum(jnp.where(lanes == support - 1, tau, 0.0), axis=1,
                       keepdims=True)
    p = jnp.where(valid, jnp.square(jnp.clip(X - tau_star, 0.0, None)), 0.0)
    p15 = p * jnp.sqrt(p)
    omega = (1.0 - jnp.sum(p15, axis=1)) / 0.75
    dot = jnp.sum(p * jnp.where(valid, v, 0.0), axis=1)
    loss = omega + dot - zt_ref[:, 0]
    out_ref[...] = (jnp.sum(loss) / float(N)).reshape(1, 1)


def _tail(cand, zt):
    out = pl.pallas_call(
        _tail_body,
        out_shape=jax.ShapeDtypeStruct((1, 1), jnp.float32),
    )(cand, zt.reshape(N, 1))
    return out[0, 0]


def kernel(input, target):
    cand, zt = _sc_topk(input, target)
    return _tail(cand, zt)


# speedup vs baseline: 13.2332x; 2.1988x over previous
import functools

import jax
import jax.numpy as jnp
import numpy as np
from jax import lax
from jax.experimental import pallas as pl
from jax.experimental.pallas import tpu as pltpu
from jax.experimental.pallas import tpu_sc as plsc

N, V, K = 1024, 100000, 100
CW = 128
NC, NS, L = 2, 16, 16
NW = NC * NS
RPW = N // NW
CHUNK = 20000
NCHUNK = V // CHUNK
SEGV = 50
UNROLL = 10
NSEG = CHUNK // (SEGV * L)
CAP = 2048
TRIGGER = 1024
INT_MIN = np.int32(-2147483648)
NEG_INF = np.float32(-np.inf)


def _iota16():
    return lax.iota(jnp.int32, L)


def _splat_f(x):
    return jnp.full((L,), x, jnp.float32)


def _splat_i(x):
    return jnp.full((L,), x, jnp.int32)


def _scalar(v):
    return jnp.max(v)


def _keymap(v):
    b = plsc.bitcast(v, jnp.int32)
    return jnp.where(b >= 0, b, jnp.bitwise_xor(jnp.bitwise_not(b), INT_MIN))


def _inv_keymap_splat(t):
    ts = _splat_i(t)
    bits = jnp.where(ts >= 0, ts, jnp.bitwise_not(jnp.bitwise_xor(ts, INT_MIN)))
    return plsc.bitcast(bits, jnp.float32)


def _count_ge(keybuf, nv4, cand):
    cs = _splat_i(cand)

    @plsc.parallel_loop(0, nv4 * 4, unroll=4, carry=_splat_i(0))
    def acc(j, acc):
        kv = keybuf[pl.ds(j * L, L)]
        return acc + plsc.all_reduce_population_count(kv >= cs)

    return _scalar(acc)


def _find_kth_key(candbuf, keybuf, pos_s):
    nv4 = (pos_s + (4 * L - 1)) // (4 * L)
    io = _iota16()

    @plsc.parallel_loop(0, nv4 * 4, unroll=4, carry=jnp.int32(0))
    def _fill(j, c):
        v = candbuf[pl.ds(j * L, L)]
        k = _keymap(v)
        valid = (j * L + io) < pos_s
        keybuf[pl.ds(j * L, L)] = jnp.where(valid, k, INT_MIN)
        return c

    c0 = _count_ge(keybuf, nv4, jnp.int32(0))
    t = jnp.where(c0 >= K, jnp.int32(0), INT_MIN)

    def probe(i, t):
        bit = jnp.int32(30) - i
        cand = jnp.bitwise_or(t, lax.shift_left(jnp.int32(1), bit))
        c = _count_ge(keybuf, nv4, cand)
        return jnp.where(c >= K, cand, t)

    return lax.fori_loop(0, 31, probe, t)


def _compact_gt(src_ref, dst_ref, pos_s, thr_f):
    io = _iota16()
    nv2 = (pos_s + (2 * L - 1)) // (2 * L)

    @plsc.parallel_loop(0, nv2 * 2, unroll=2, carry=_splat_i(0))
    def pos(j, pos):
        v = src_ref[pl.ds(j * L, L)]
        m = (v > thr_f) & ((j * L + io) < pos_s)
        pref = plsc.cumsum(m.astype(jnp.int32))
        idx = pos + pref - 1
        plsc.store_scatter(dst_ref, [idx], v, mask=m)
        return pos + plsc.all_reduce_population_count(m)

    return _scalar(pos)


def _sc_body(input_hbm, target_hbm, cand_hbm, zt_hbm,
             databuf, candbuf, keybuf, outstage, tgtstage, ztstage, dsem):
    wid = lax.axis_index("s") * NC + lax.axis_index("c")
    base = wid * RPW
    io = _iota16()

    pltpu.sync_copy(target_hbm.at[pl.ds(base, RPW)], tgtstage)

    def do_reselect(pos_s):
        t = _find_kth_key(candbuf, keybuf, pos_s)
        thr_f = _inv_keymap_splat(t)
        c = _compact_gt(candbuf, candbuf, pos_s, thr_f)
        for m in range(7):
            idx = c + m * L + io
            plsc.store_scatter(candbuf, [idx], thr_f, mask=idx < K)
        return thr_f, _splat_i(K)

    def row_body(rl, zt_carry):
        row = base + rl
        tg = tgtstage[pl.ds((rl // L) * L, L)]
        t_r = jnp.sum(jnp.where(io == (rl % L), tg, 0))
        tchunk = t_r // CHUNK

        def chunk_body(c, carry):
            thr_f, pos, ztv = carry
            pltpu.sync_copy(input_hbm.at[row, pl.ds(c * CHUNK, CHUNK)],
                            databuf)

            def seg_body(s, carry2):
                thr_f, pos = carry2

                @plsc.parallel_loop(s * SEGV, (s + 1) * SEGV,
                                    unroll=UNROLL, carry=pos)
                def pos(i, pos):
                    v = databuf[pl.ds(i * L, L)]
                    m = v > thr_f
                    pref = plsc.cumsum(m.astype(jnp.int32))
                    idx = pos + pref - 1
                    plsc.store_scatter(candbuf, [idx], v, mask=m)
                    return pos + plsc.all_reduce_population_count(m)
                pos_s = _scalar(pos)
                thr_f, pos = lax.cond(pos_s > TRIGGER,
                                      lambda: do_reselect(pos_s),
                                      lambda: (thr_f, pos))
                return thr_f, pos

            thr_f, pos = lax.fori_loop(0, NSEG, seg_body, (thr_f, pos))

            o = jnp.clip(t_r - c * CHUNK, 0, CHUNK - 1)
            al = (o // L) * L
            v16 = databuf[pl.ds(al, L)]
            val = jnp.sum(jnp.where(io == (o - al), v16, jnp.float32(0.0)))
            hit = (tchunk == c).astype(jnp.float32)
            ztv = ztv + hit * val * jnp.where(io == (rl % L), 1.0, 0.0)
            return thr_f, pos, ztv

        thr_f, pos, zt_carry = lax.fori_loop(
            0, NCHUNK, chunk_body,
            (_splat_f(NEG_INF), _splat_i(0), zt_carry))

        pos_s = _scalar(pos)
        t100 = _find_kth_key(candbuf, keybuf, pos_s)
        tf = _inv_keymap_splat(t100)
        for m in range(CW // L):
            outstage[pl.ds(m * L, L)] = jnp.where(m * L + io < K, tf, NEG_INF)
        cnt = _compact_gt(candbuf, outstage, pos_s, tf)
        del cnt
        pltpu.sync_copy(outstage, cand_hbm.at[row])
        return zt_carry

    zt = lax.fori_loop(0, L, row_body, _splat_f(0.0))
    ztstage[pl.ds(0, L)] = zt
    zt2 = lax.fori_loop(L, RPW, row_body, _splat_f(0.0))
    ztstage[pl.ds(L, L)] = zt2
    pltpu.sync_copy(ztstage, zt_hbm.at[pl.ds(base, RPW)])


@functools.partial(jax.jit, static_argnums=())
def _sc_topk(input, target):
    mesh = plsc.VectorSubcoreMesh(core_axis_name="c", subcore_axis_name="s",
                                  num_cores=NC, num_subcores=NS)
    f = pl.kernel(
        _sc_body,
        out_type=(
            jax.ShapeDtypeStruct((N, CW), jnp.float32),
            jax.ShapeDtypeStruct((N,), jnp.float32),
        ),
        mesh=mesh,
        scratch_types=[
            pltpu.VMEM((CHUNK,), jnp.float32),
            pltpu.VMEM((CAP,), jnp.float32),
            pltpu.VMEM((CAP,), jnp.int32),
            pltpu.VMEM((CW,), jnp.float32),
            pltpu.VMEM((RPW,), jnp.int32),
            pltpu.VMEM((RPW,), jnp.float32),
            pltpu.SemaphoreType.DMA,
        ],
        compiler_params=pltpu.CompilerParams(use_tc_tiling_on_sc=False,
                                             needs_layout_passes=False),
    )
    return f(input, target)


def _tail_body(cand_ref, zt_ref, out_ref):
    v = cand_ref[...]
    lanes = lax.broadcasted_iota(jnp.int32, v.shape, 1)

    def rolled(x, s):
        left = jnp.concatenate([x[:, s:], x[:, :s]], axis=1)
        right = jnp.concatenate([x[:, -s:], x[:, :-s]], axis=1)
        return jnp.where((lanes & s) == 0, left, right)

    k = 2
    while k <= CW:
        j = k // 2
        while j >= 1:
            p = rolled(v, j)
            take_max = ((lanes & k) == 0) == ((lanes & j) == 0)
            v = jnp.where(take_max, jnp.maximum(v, p), jnp.minimum(v, p))
            j //= 2
        k *= 2

    X = v * 0.5
    valid = lanes < K
    Xs = jnp.where(valid, X, 0.0)
    cum1 = Xs
    cum2 = Xs * Xs
    s = 1
    while s < CW:
        def shr(x, sh):
            return jnp.concatenate(
                [jnp.zeros((x.shape[0], sh), x.dtype), x[:, :-sh]], axis=1)
        cum1 = cum1 + shr(cum1, s)
        cum2 = cum2 + shr(cum2, s)
        s *= 2

    rho = (lanes + 1).astype(jnp.float32)
    mean = cum1 / rho
    meansq = cum2 / rho
    ss = rho * (meansq - mean * mean)
    delta = (1.0 - ss) / rho
    tau = mean - jnp.sqrt(jnp.clip(delta, 0.0, None))
    support_mask = (tau <= X) & valid
    support = jnp.sum(support_mask.astype(jnp.int32), axis=1, keepdims=True)
    tau_star = jnp.sum(jnp.where(lanes == support - 1, tau, 0.0), axis=1,
                       keepdims=True)
    p = jnp.where(valid, jnp.square(jnp.clip(X - tau_star, 0.0, None)), 0.0)
    p15 = p * jnp.sqrt(p)
    omega = (1.0 - jnp.sum(p15, axis=1)) / 0.75
    dot = jnp.sum(p * jnp.where(valid, v, 0.0), axis=1)
    loss = omega + dot - zt_ref[:, 0]
    out_ref[...] = (jnp.sum(loss) / float(N)).reshape(1, 1)


def _tail(cand, zt):
    out = pl.pallas_call(
        _tail_body,
        out_shape=jax.ShapeDtypeStruct((1, 1), jnp.float32),
    )(cand, zt.reshape(N, 1))
    return out[0, 0]


def kernel(input, target):
    cand, zt = _sc_topk(input, target)
    return _tail(cand, zt)


# speedup vs baseline: 18.5741x; 1.4036x over previous
import functools

import jax
import jax.numpy as jnp
import numpy as np
from jax import lax
from jax.experimental import pallas as pl
from jax.experimental.pallas import tpu as pltpu
from jax.experimental.pallas import tpu_sc as plsc

N, V, K = 1024, 100000, 100
CW = 128
NC, NS, L = 2, 16, 16
NW = NC * NS
RPW = N // NW
GROUPS = RPW // 8
CHUNK = 6400
NFULL = 15
V_SC = 99968
TAIL = V_SC - NFULL * CHUNK
TAIL_SEGS = 4
TAIL_XVEC = 48
VX = V - V_SC
SEGV = 50
UNROLL = 10
CAP = 2048
TRIGGER = 1024
INT_MIN = np.int32(-2147483648)
NEG_INF = np.float32(-np.inf)


def _iota16():
    return lax.iota(jnp.int32, L)


def _splat_f(x):
    return jnp.full((L,), x, jnp.float32)


def _splat_i(x):
    return jnp.full((L,), x, jnp.int32)


def _scalar(v):
    return jnp.max(v)


def _keymap(v):
    b = plsc.bitcast(v, jnp.int32)
    return jnp.where(b >= 0, b, jnp.bitwise_xor(jnp.bitwise_not(b), INT_MIN))


def _inv_keymap_splat(t):
    ts = _splat_i(t)
    bits = jnp.where(ts >= 0, ts, jnp.bitwise_not(jnp.bitwise_xor(ts, INT_MIN)))
    return plsc.bitcast(bits, jnp.float32)


def _count_ge(keybuf, nv4, cand):
    cs = _splat_i(cand)

    @plsc.parallel_loop(0, nv4 * 4, unroll=4, carry=_splat_i(0))
    def acc(j, acc):
        kv = keybuf[pl.ds(j * L, L)]
        return acc + plsc.all_reduce_population_count(kv >= cs)

    return _scalar(acc)


def _find_kth_key(candbuf, keybuf, pos_s):
    nv4 = (pos_s + (4 * L - 1)) // (4 * L)
    io = _iota16()

    @plsc.parallel_loop(0, nv4 * 4, unroll=4, carry=jnp.int32(0))
    def _fill(j, c):
        v = candbuf[pl.ds(j * L, L)]
        k = _keymap(v)
        valid = (j * L + io) < pos_s
        keybuf[pl.ds(j * L, L)] = jnp.where(valid, k, INT_MIN)
        return c

    c0 = _count_ge(keybuf, nv4, jnp.int32(0))
    t = jnp.where(c0 >= K, jnp.int32(0), INT_MIN)

    def probe(i, t):
        bit = jnp.int32(30) - i
        cand = jnp.bitwise_or(t, lax.shift_left(jnp.int32(1), bit))
        c = _count_ge(keybuf, nv4, cand)
        return jnp.where(c >= K, cand, t)

    return lax.fori_loop(0, 31, probe, t)


def _compact_gt(src_ref, dst_ref, pos_s, thr_f):
    io = _iota16()
    nv2 = (pos_s + (2 * L - 1)) // (2 * L)

    @plsc.parallel_loop(0, nv2 * 2, unroll=2, carry=_splat_i(0))
    def pos(j, pos):
        v = src_ref[pl.ds(j * L, L)]
        m = (v > thr_f) & ((j * L + io) < pos_s)
        pref = plsc.cumsum(m.astype(jnp.int32))
        idx = pos + pref - 1
        plsc.store_scatter(dst_ref, [idx], v, mask=m)
        return pos + plsc.all_reduce_population_count(m)

    return _scalar(pos)


def _sc_body(input_hbm, target_hbm, cand_hbm, zt_hbm,
             databuf, tailbuf, candbuf, keybuf, outstage, tgtstage, ztstage,
             dsem):
    wid = lax.axis_index("s") * NC + lax.axis_index("c")
    base = wid * RPW
    io = _iota16()

    pltpu.sync_copy(target_hbm.at[pl.ds(base, RPW)], tgtstage)

    def do_reselect(rbase, pos_s):
        area = candbuf.at[pl.ds(rbase, CAP)]
        t = _find_kth_key(area, keybuf, pos_s)
        thr_f = _inv_keymap_splat(t)
        c = _compact_gt(area, area, pos_s, thr_f)
        for m in range(7):
            idx = c + m * L + io
            plsc.store_scatter(area, [idx], thr_f, mask=idx < K)
        return thr_f, _splat_i(K)

    def process_chunk(buf, nseg, xvec, ccol, clen, g, carry):
        def row_body(r, carry):
            thr8, pos8, zt0, zt1 = carry
            thr_f = _splat_f(jnp.max(jnp.where(io == r, thr8, NEG_INF)))
            pos = _splat_i(jnp.max(jnp.where(io == r, pos8, 0)))
            rbase = r * CAP

            def scan_block(lo, hi, unroll, thr_f, pos):
                @plsc.parallel_loop(lo, hi, unroll=unroll, carry=pos)
                def pos(i, pos):
                    v = buf[r, pl.ds(i * L, L)]
                    m = v > thr_f
                    pref = plsc.cumsum(m.astype(jnp.int32))
                    idx = rbase + (pos + pref - 1)
                    plsc.store_scatter(candbuf, [idx], v, mask=m)
                    return pos + plsc.all_reduce_population_count(m)

                pos_s = _scalar(pos)
                return lax.cond(pos_s > TRIGGER,
                                lambda: do_reselect(rbase, pos_s),
                                lambda: (thr_f, pos))

            def seg_body(s, carry2):
                thr_f, pos = carry2
                return scan_block(s * SEGV, (s + 1) * SEGV, UNROLL,
                                  thr_f, pos)

            thr_f, pos = lax.fori_loop(0, nseg, seg_body, (thr_f, pos))
            if xvec:
                thr_f, pos = scan_block(nseg * SEGV, nseg * SEGV + xvec, 8,
                                        thr_f, pos)

            rl = g * 8 + r
            tg = tgtstage[pl.ds((rl // L) * L, L)]
            t_r = jnp.sum(jnp.where(io == (rl % L), tg, 0))
            o = t_r - ccol
            hit = (o >= 0) & (o < clen)
            oc = jnp.clip(o, 0, clen - 1)
            al = (oc // L) * L
            v16 = buf[r, pl.ds(al, L)]
            val = jnp.sum(jnp.where(io == (oc - al), v16, jnp.float32(0.0)))
            upd = hit.astype(jnp.float32) * val * \
                jnp.where(io == (rl % L), 1.0, 0.0)
            zt0 = zt0 + jnp.where(rl < L, upd, 0.0)
            zt1 = zt1 + jnp.where(rl >= L, upd, 0.0)

            thr8 = jnp.where(io == r, thr_f, thr8)
            pos8 = jnp.where(io == r, pos, pos8)
            return thr8, pos8, zt0, zt1

        return lax.fori_loop(0, 8, row_body, carry)

    def group_body(g, zt_carry):
        zt0, zt1 = zt_carry
        grow = base + g * 8

        def chunk_body(c, ch_carry):
            pltpu.sync_copy(
                input_hbm.at[pl.ds(grow, 8), pl.ds(c * CHUNK, CHUNK)],
                databuf)
            return process_chunk(databuf, CHUNK // (SEGV * L), 0,
                                 c * CHUNK, CHUNK, g, ch_carry)

        carry = (_splat_f(NEG_INF), _splat_i(0), zt0, zt1)
        carry = lax.fori_loop(0, NFULL, chunk_body, carry)

        pltpu.sync_copy(
            input_hbm.at[pl.ds(grow, 8), pl.ds(NFULL * CHUNK, TAIL)],
            tailbuf)
        thr8, pos8, zt0, zt1 = process_chunk(
            tailbuf, TAIL_SEGS, TAIL_XVEC, NFULL * CHUNK, TAIL, g, carry)

        def fin_body(r, c):
            pos_s = jnp.max(jnp.where(io == r, pos8, 0))
            area = candbuf.at[pl.ds(r * CAP, CAP)]
            t100 = _find_kth_key(area, keybuf, pos_s)
            tf = _inv_keymap_splat(t100)
            orow = outstage.at[r]
            for m in range(CW // L):
                orow[pl.ds(m * L, L)] = jnp.where(m * L + io < K, tf, NEG_INF)
            _compact_gt(area, orow, pos_s, tf)
            return c

        lax.fori_loop(0, 8, fin_body, 0)
        pltpu.sync_copy(outstage, cand_hbm.at[pl.ds(grow, 8), :])
        return zt0, zt1

    zt0, zt1 = lax.fori_loop(0, GROUPS, group_body,
                             (_splat_f(0.0), _splat_f(0.0)))
    ztstage[pl.ds(0, L)] = zt0
    ztstage[pl.ds(L, L)] = zt1
    pltpu.sync_copy(ztstage, zt_hbm.at[pl.ds(base, RPW)])


@functools.partial(jax.jit, static_argnums=())
def _sc_topk(input, target):
    mesh = plsc.VectorSubcoreMesh(core_axis_name="c", subcore_axis_name="s",
                                  num_cores=NC, num_subcores=NS)
    f = pl.kernel(
        _sc_body,
        out_type=(
            jax.ShapeDtypeStruct((N, CW), jnp.float32),
            jax.ShapeDtypeStruct((N,), jnp.float32),
        ),
        mesh=mesh,
        scratch_types=[
            pltpu.VMEM((8, CHUNK), jnp.float32),
            pltpu.VMEM((8, TAIL), jnp.float32),
            pltpu.VMEM((8 * CAP,), jnp.float32),
            pltpu.VMEM((CAP,), jnp.int32),
            pltpu.VMEM((8, CW), jnp.float32),
            pltpu.VMEM((RPW,), jnp.int32),
            pltpu.VMEM((RPW,), jnp.float32),
            pltpu.SemaphoreType.DMA,
        ],
        compiler_params=pltpu.CompilerParams(needs_layout_passes=False),
    )
    return f(input, target)


def _tail_body(cand_ref, extra_ref, zt_ref, tgt_ref, out_ref):
    v = cand_ref[...]
    lanes = lax.broadcasted_iota(jnp.int32, v.shape, 1)

    def rolled(x, s):
        left = jnp.concatenate([x[:, s:], x[:, :s]], axis=1)
        right = jnp.concatenate([x[:, -s:], x[:, :-s]], axis=1)
        return jnp.where((lanes & s) == 0, left, right)

    def bitonic(x, descending):
        k = 2
        while k <= CW:
            j = k // 2
            while j >= 1:
                p = rolled(x, j)
                take_max = ((lanes & k) == 0) == ((lanes & j) == 0)
                if not descending:
                    take_max = ~take_max
                x = jnp.where(take_max, jnp.maximum(x, p), jnp.minimum(x, p))
                j //= 2
            k *= 2
        return x

    ex = jnp.where(lanes < VX, extra_ref[...], NEG_INF)
    tl = tgt_ref[...] - V_SC
    ztfix = jnp.sum(jnp.where(lanes == tl, ex, 0.0), axis=1)

    va = bitonic(v, True)
    vb = bitonic(ex, False)
    vm = jnp.maximum(va, vb)
    j = CW // 2
    while j >= 1:
        p = rolled(vm, j)
        take_max = (lanes & j) == 0
        vm = jnp.where(take_max, jnp.maximum(vm, p), jnp.minimum(vm, p))
        j //= 2
    v = vm

    X = v * 0.5
    valid = lanes < K
    Xs = jnp.where(valid, X, 0.0)
    cum1 = Xs
    cum2 = Xs * Xs
    s = 1
    while s < CW:
        def shr(x, sh):
            return jnp.concatenate(
                [jnp.zeros((x.shape[0], sh), x.dtype), x[:, :-sh]], axis=1)
        cum1 = cum1 + shr(cum1, s)
        cum2 = cum2 + shr(cum2, s)
        s *= 2

    rho = (lanes + 1).astype(jnp.float32)
    mean = cum1 / rho
    meansq = cum2 / rho
    ss = rho * (meansq - mean * mean)
    delta = (1.0 - ss) / rho
    tau = mean - jnp.sqrt(jnp.clip(delta, 0.0, None))
    support_mask = (tau <= X) & valid
    support = jnp.sum(support_mask.astype(jnp.int32), axis=1, keepdims=True)
    tau_star = jnp.sum(jnp.where(lanes == support - 1, tau, 0.0), axis=1,
                       keepdims=True)
    p = jnp.where(valid, jnp.square(jnp.clip(X - tau_star, 0.0, None)), 0.0)
    p15 = p * jnp.sqrt(p)
    omega = (1.0 - jnp.sum(p15, axis=1)) / 0.75
    dot = jnp.sum(p * jnp.where(valid, v, 0.0), axis=1)
    loss = omega + dot - zt_ref[:, 0] - ztfix
    out_ref[...] = (jnp.sum(loss) / float(N)).reshape(1, 1)


def _tail(cand, input, zt, target):
    out = pl.pallas_call(
        _tail_body,
        grid=(1,),
        in_specs=[
            pl.BlockSpec((N, CW), lambda i: (0, 0)),
            pl.BlockSpec((N, CW), lambda i: (0, V_SC // CW)),
            pl.BlockSpec((N, 1), lambda i: (0, 0)),
            pl.BlockSpec((N, 1), lambda i: (0, 0)),
        ],
        out_specs=pl.BlockSpec((1, 1), lambda i: (0, 0)),
        out_shape=jax.ShapeDtypeStruct((1, 1), jnp.float32),
    )(cand, input, zt.reshape(N, 1), target.reshape(N, 1))
    return out[0, 0]


def kernel(input, target):
    cand, zt = _sc_topk(input, target)
    return _tail(cand, input, zt, target)


# speedup vs baseline: 18.5786x; 1.0002x over previous
import functools

import jax
import jax.numpy as jnp
import numpy as np
from jax import lax
from jax.experimental import pallas as pl
from jax.experimental.pallas import tpu as pltpu
from jax.experimental.pallas import tpu_sc as plsc

N, V, K = 1024, 100000, 100
CW = 128
NC, NS, L = 2, 16, 16
NW = NC * NS
RPW = N // NW
GROUPS = RPW // 8
CHUNK = 6400
NFULL = 15
V_SC = 99968
TAIL = V_SC - NFULL * CHUNK
TAIL_SEGS = 4
TAIL_XVEC = 48
VX = V - V_SC
SEGV = 50
UNROLL = 10
CAP = 2048
TRIGGER = 1024
INT_MIN = np.int32(-2147483648)
NEG_INF = np.float32(-np.inf)


def _iota16():
    return lax.iota(jnp.int32, L)


def _splat_f(x):
    return jnp.full((L,), x, jnp.float32)


def _splat_i(x):
    return jnp.full((L,), x, jnp.int32)


def _scalar(v):
    return jnp.max(v)


def _keymap(v):
    b = plsc.bitcast(v, jnp.int32)
    return jnp.where(b >= 0, b, jnp.bitwise_xor(jnp.bitwise_not(b), INT_MIN))


def _inv_keymap_splat(t):
    ts = _splat_i(t)
    bits = jnp.where(ts >= 0, ts, jnp.bitwise_not(jnp.bitwise_xor(ts, INT_MIN)))
    return plsc.bitcast(bits, jnp.float32)


def _count_ge(keybuf, nv4, cand):
    cs = _splat_i(cand)

    @plsc.parallel_loop(0, nv4 * 4, unroll=4, carry=_splat_i(0))
    def acc(j, acc):
        kv = keybuf[pl.ds(j * L, L)]
        return acc + plsc.all_reduce_population_count(kv >= cs)

    return _scalar(acc)


def _find_kth_key(candbuf, keybuf, pos_s):
    nv4 = (pos_s + (4 * L - 1)) // (4 * L)
    io = _iota16()

    @plsc.parallel_loop(0, nv4 * 4, unroll=4, carry=jnp.int32(0))
    def _fill(j, c):
        v = candbuf[pl.ds(j * L, L)]
        k = _keymap(v)
        valid = (j * L + io) < pos_s
        keybuf[pl.ds(j * L, L)] = jnp.where(valid, k, INT_MIN)
        return c

    c0 = _count_ge(keybuf, nv4, jnp.int32(0))
    t = jnp.where(c0 >= K, jnp.int32(0), INT_MIN)

    def probe(i, t):
        bit = jnp.int32(30) - i
        cand = jnp.bitwise_or(t, lax.shift_left(jnp.int32(1), bit))
        c = _count_ge(keybuf, nv4, cand)
        return jnp.where(c >= K, cand, t)

    return lax.fori_loop(0, 31, probe, t)


def _compact_gt(src_ref, dst_ref, pos_s, thr_f):
    io = _iota16()
    nv2 = (pos_s + (2 * L - 1)) // (2 * L)

    @plsc.parallel_loop(0, nv2 * 2, unroll=2, carry=_splat_i(0))
    def pos(j, pos):
        v = src_ref[pl.ds(j * L, L)]
        m = (v > thr_f) & ((j * L + io) < pos_s)
        pref = plsc.cumsum(m.astype(jnp.int32))
        idx = pos + pref - 1
        plsc.store_scatter(dst_ref, [idx], v, mask=m)
        return pos + plsc.all_reduce_population_count(m)

    return _scalar(pos)


def _sc_body(input_hbm, target_hbm, cand_hbm, zt_hbm,
             databuf, tailbuf, candbuf, keybuf, outstage, tgtstage, ztstage,
             dsem):
    wid = lax.axis_index("s") * NC + lax.axis_index("c")
    base = wid * RPW
    io = _iota16()

    pltpu.sync_copy(target_hbm.at[pl.ds(base, RPW)], tgtstage)

    def do_reselect(rbase, pos_s):
        area = candbuf.at[pl.ds(rbase, CAP)]
        t = _find_kth_key(area, keybuf, pos_s)
        thr_f = _inv_keymap_splat(t)
        c = _compact_gt(area, area, pos_s, thr_f)
        for m in range(7):
            idx = c + m * L + io
            plsc.store_scatter(area, [idx], thr_f, mask=idx < K)
        return thr_f, _splat_i(K)

    def process_chunk(buf, nseg, xvec, ccol, clen, g, carry):
        def row_body(r, carry):
            thr8, pos8, zt0, zt1 = carry
            thr_f = _splat_f(jnp.max(jnp.where(io == r, thr8, NEG_INF)))
            pos = _splat_i(jnp.max(jnp.where(io == r, pos8, 0)))
            rbase = r * CAP

            def scan_block(lo, hi, unroll, thr_f, pos):
                @plsc.parallel_loop(lo, hi, unroll=unroll, carry=pos)
                def pos(i, pos):
                    v = buf[r, pl.ds(i * L, L)]
                    m = v > thr_f
                    pref = plsc.cumsum(m.astype(jnp.int32))
                    idx = rbase + (pos + pref - 1)
                    plsc.store_scatter(candbuf, [idx], v, mask=m)
                    return pos + plsc.all_reduce_population_count(m)

                pos_s = _scalar(pos)
                return lax.cond(pos_s > TRIGGER,
                                lambda: do_reselect(rbase, pos_s),
                                lambda: (thr_f, pos))

            def seg_body(s, carry2):
                thr_f, pos = carry2
                return scan_block(s * SEGV, (s + 1) * SEGV, UNROLL,
                                  thr_f, pos)

            thr_f, pos = lax.fori_loop(0, nseg, seg_body, (thr_f, pos))
            if xvec:
                thr_f, pos = scan_block(nseg * SEGV, nseg * SEGV + xvec, 8,
                                        thr_f, pos)

            rl = g * 8 + r
            tg = tgtstage[pl.ds((rl // L) * L, L)]
            t_r = jnp.sum(jnp.where(io == (rl % L), tg, 0))
            o = t_r - ccol
            hit = (o >= 0) & (o < clen)
            oc = jnp.clip(o, 0, clen - 1)
            al = (oc // L) * L
            v16 = buf[r, pl.ds(al, L)]
            val = jnp.sum(jnp.where(io == (oc - al), v16, jnp.float32(0.0)))
            upd = hit.astype(jnp.float32) * val * \
                jnp.where(io == (rl % L), 1.0, 0.0)
            zt0 = zt0 + jnp.where(rl < L, upd, 0.0)
            zt1 = zt1 + jnp.where(rl >= L, upd, 0.0)

            thr8 = jnp.where(io == r, thr_f, thr8)
            pos8 = jnp.where(io == r, pos, pos8)
            return thr8, pos8, zt0, zt1

        return lax.fori_loop(0, 8, row_body, carry)

    def group_body(g, zt_carry):
        zt0, zt1 = zt_carry
        grow = base + g * 8

        def chunk_body(c, ch_carry):
            pltpu.sync_copy(
                input_hbm.at[pl.ds(grow, 8), pl.ds(c * CHUNK, CHUNK)],
                databuf)
            return process_chunk(databuf, CHUNK // (SEGV * L), 0,
                                 c * CHUNK, CHUNK, g, ch_carry)

        carry = (_splat_f(NEG_INF), _splat_i(0), zt0, zt1)
        carry = lax.fori_loop(0, NFULL, chunk_body, carry)

        pltpu.sync_copy(
            input_hbm.at[pl.ds(grow, 8), pl.ds(NFULL * CHUNK, TAIL)],
            tailbuf)
        thr8, pos8, zt0, zt1 = process_chunk(
            tailbuf, TAIL_SEGS, TAIL_XVEC, NFULL * CHUNK, TAIL, g, carry)

        def fin_body(r, c):
            pos_s = jnp.max(jnp.where(io == r, pos8, 0))
            area = candbuf.at[pl.ds(r * CAP, CAP)]
            t100 = _find_kth_key(area, keybuf, pos_s)
            tf = _inv_keymap_splat(t100)
            orow = outstage.at[r]
            for m in range(CW // L):
                orow[pl.ds(m * L, L)] = jnp.where(m * L + io < K, tf, NEG_INF)
            _compact_gt(area, orow, pos_s, tf)
            return c

        lax.fori_loop(0, 8, fin_body, 0)
        pltpu.sync_copy(outstage, cand_hbm.at[pl.ds(grow, 8), :])
        return zt0, zt1

    zt0, zt1 = lax.fori_loop(0, GROUPS, group_body,
                             (_splat_f(0.0), _splat_f(0.0)))
    ztstage[pl.ds(0, L)] = zt0
    ztstage[pl.ds(L, L)] = zt1
    pltpu.sync_copy(ztstage, zt_hbm.at[pl.ds(base, RPW)])


@functools.partial(jax.jit, static_argnums=())
def _sc_topk(input, target):
    mesh = plsc.VectorSubcoreMesh(core_axis_name="c", subcore_axis_name="s",
                                  num_cores=NC, num_subcores=NS)
    f = pl.kernel(
        _sc_body,
        out_type=(
            jax.ShapeDtypeStruct((N, CW), jnp.float32),
            jax.ShapeDtypeStruct((N,), jnp.float32),
        ),
        mesh=mesh,
        scratch_types=[
            pltpu.VMEM((8, CHUNK), jnp.float32),
            pltpu.VMEM((8, TAIL), jnp.float32),
            pltpu.VMEM((8 * CAP,), jnp.float32),
            pltpu.VMEM((CAP,), jnp.int32),
            pltpu.VMEM((8, CW), jnp.float32),
            pltpu.VMEM((RPW,), jnp.int32),
            pltpu.VMEM((RPW,), jnp.float32),
            pltpu.SemaphoreType.DMA,
        ],
        compiler_params=pltpu.CompilerParams(needs_layout_passes=False),
    )
    return f(input, target)


def _tail_body(cand_ref, extra_ref, zt_ref, tgt_ref, out_ref):
    v = cand_ref[...]
    lanes = lax.broadcasted_iota(jnp.int32, v.shape, 1)

    def rolled(x, s):
        left = jnp.concatenate([x[:, s:], x[:, :s]], axis=1)
        right = jnp.concatenate([x[:, -s:], x[:, :-s]], axis=1)
        return jnp.where((lanes & s) == 0, left, right)

    def bitonic(x, descending):
        k = 2
        while k <= CW:
            j = k // 2
            while j >= 1:
                p = rolled(x, j)
                take_max = ((lanes & k) == 0) == ((lanes & j) == 0)
                if not descending:
                    take_max = ~take_max
                x = jnp.where(take_max, jnp.maximum(x, p), jnp.minimum(x, p))
                j //= 2
            k *= 2
        return x

    ex = jnp.concatenate(
        [extra_ref[...], jnp.full((v.shape[0], CW - VX), NEG_INF, v.dtype)],
        axis=1)
    tl = tgt_ref[...] - V_SC
    ztfix = jnp.sum(jnp.where(lanes == tl, ex, 0.0), axis=1)

    va = bitonic(v, True)
    vb = bitonic(ex, False)
    vm = jnp.maximum(va, vb)
    j = CW // 2
    while j >= 1:
        p = rolled(vm, j)
        take_max = (lanes & j) == 0
        vm = jnp.where(take_max, jnp.maximum(vm, p), jnp.minimum(vm, p))
        j //= 2
    v = vm

    X = v * 0.5
    valid = lanes < K
    Xs = jnp.where(valid, X, 0.0)
    cum1 = Xs
    cum2 = Xs * Xs
    s = 1
    while s < CW:
        def shr(x, sh):
            return jnp.concatenate(
                [jnp.zeros((x.shape[0], sh), x.dtype), x[:, :-sh]], axis=1)
        cum1 = cum1 + shr(cum1, s)
        cum2 = cum2 + shr(cum2, s)
        s *= 2

    rho = (lanes + 1).astype(jnp.float32)
    mean = cum1 / rho
    meansq = cum2 / rho
    ss = rho * (meansq - mean * mean)
    delta = (1.0 - ss) / rho
    tau = mean - jnp.sqrt(jnp.clip(delta, 0.0, None))
    support_mask = (tau <= X) & valid
    support = jnp.sum(support_mask.astype(jnp.int32), axis=1, keepdims=True)
    tau_star = jnp.sum(jnp.where(lanes == support - 1, tau, 0.0), axis=1,
                       keepdims=True)
    p = jnp.where(valid, jnp.square(jnp.clip(X - tau_star, 0.0, None)), 0.0)
    p15 = p * jnp.sqrt(p)
    omega = (1.0 - jnp.sum(p15, axis=1)) / 0.75
    dot = jnp.sum(p * jnp.where(valid, v, 0.0), axis=1)
    loss = omega + dot - zt_ref[:, 0] - ztfix
    out_ref[...] = (jnp.sum(loss) / float(N)).reshape(1, 1)


def _tail(cand, extra, zt, target):
    out = pl.pallas_call(
        _tail_body,
        grid=(1,),
        in_specs=[
            pl.BlockSpec((N, CW), lambda i: (0, 0)),
            pl.BlockSpec((N, VX), lambda i: (0, 0)),
            pl.BlockSpec((N, 1), lambda i: (0, 0)),
            pl.BlockSpec((N, 1), lambda i: (0, 0)),
        ],
        out_specs=pl.BlockSpec((1, 1), lambda i: (0, 0)),
        out_shape=jax.ShapeDtypeStruct((1, 1), jnp.float32),
    )(cand, extra, zt.reshape(N, 1), target.reshape(N, 1))
    return out[0, 0]


def kernel(input, target):
    cand, zt = _sc_topk(input, target)
    extra = lax.slice(input, (0, V_SC), (N, V))
    return _tail(cand, extra, zt, target)


# speedup vs baseline: 22.0720x; 1.1880x over previous
import functools

import jax
import jax.numpy as jnp
import numpy as np
from jax import lax
from jax.experimental import pallas as pl
from jax.experimental.pallas import tpu as pltpu
from jax.experimental.pallas import tpu_sc as plsc

N, V, K = 1024, 100000, 100
CW = 128
NC, NS, L = 2, 16, 16
NW = NC * NS
RPW = N // NW
GROUPS = RPW // 8
CHUNK = 3200
NFULL = 31
V_SC = 99968
TAIL = V_SC - NFULL * CHUNK
TAIL_SEGS = 0
TAIL_XVEC = 48
VX = V - V_SC
SEGV = 100
UNROLL = 10
CAP = 4096
TRIGGER = 2048
MIDBITS = 16
INT_MIN = np.int32(-2147483648)
NEG_INF = np.float32(-np.inf)


def _iota16():
    return lax.iota(jnp.int32, L)


def _splat_f(x):
    return jnp.full((L,), x, jnp.float32)


def _splat_i(x):
    return jnp.full((L,), x, jnp.int32)


def _scalar(v):
    return jnp.max(v)


def _keymap(v):
    b = plsc.bitcast(v, jnp.int32)
    return jnp.where(b >= 0, b, jnp.bitwise_xor(jnp.bitwise_not(b), INT_MIN))


def _inv_keymap_splat(t):
    ts = _splat_i(t)
    bits = jnp.where(ts >= 0, ts, jnp.bitwise_not(jnp.bitwise_xor(ts, INT_MIN)))
    return plsc.bitcast(bits, jnp.float32)


def _count_ge(keybuf, nv4, cand):
    cs = _splat_i(cand)

    @plsc.parallel_loop(0, nv4 * 4, unroll=4, carry=_splat_i(0))
    def acc(j, acc):
        kv = keybuf[pl.ds(j * L, L)]
        return acc + plsc.all_reduce_population_count(kv >= cs)

    return _scalar(acc)


def _find_kth_key(candbuf, keybuf, pos_s, nbits):
    nv4 = (pos_s + (4 * L - 1)) // (4 * L)
    io = _iota16()

    @plsc.parallel_loop(0, nv4 * 4, unroll=4, carry=jnp.int32(0))
    def _fill(j, c):
        v = candbuf[pl.ds(j * L, L)]
        k = _keymap(v)
        valid = (j * L + io) < pos_s
        keybuf[pl.ds(j * L, L)] = jnp.where(valid, k, INT_MIN)
        return c

    c0 = _count_ge(keybuf, nv4, jnp.int32(0))
    t = jnp.where(c0 >= K, jnp.int32(0), INT_MIN)

    def probe(i, t):
        bit = jnp.int32(30) - i
        cand = jnp.bitwise_or(t, lax.shift_left(jnp.int32(1), bit))
        c = _count_ge(keybuf, nv4, cand)
        return jnp.where(c >= K, cand, t)

    return lax.fori_loop(0, nbits, probe, t)


def _compact_gt(src_ref, dst_ref, pos_s, thr_f):
    io = _iota16()
    nv2 = (pos_s + (2 * L - 1)) // (2 * L)

    @plsc.parallel_loop(0, nv2 * 2, unroll=2, carry=_splat_i(0))
    def pos(j, pos):
        v = src_ref[pl.ds(j * L, L)]
        m = (v > thr_f) & ((j * L + io) < pos_s)
        pref = plsc.cumsum(m.astype(jnp.int32))
        idx = pos + pref - 1
        plsc.store_scatter(dst_ref, [idx], v, mask=m)
        return pos + plsc.all_reduce_population_count(m)

    return _scalar(pos)


def _sc_body(input_hbm, target_hbm, cand_hbm, zt_hbm,
             databuf, databuf2, tailbuf, candbuf, keybuf, outstage, tgtstage,
             ztstage, semA, semB):
    wid = lax.axis_index("s") * NC + lax.axis_index("c")
    base = wid * RPW
    io = _iota16()

    pltpu.sync_copy(target_hbm.at[pl.ds(base, RPW)], tgtstage)

    def do_reselect(rbase, pos_s, nbits):
        area = candbuf.at[pl.ds(rbase, CAP)]
        t = _find_kth_key(area, keybuf, pos_s, nbits)
        thr_f = _inv_keymap_splat(t)
        c = _compact_gt(area, area, pos_s, thr_f)
        for m in range(7):
            idx = c + m * L + io
            plsc.store_scatter(area, [idx], thr_f, mask=idx < K)
        return thr_f, _splat_i(jnp.maximum(c, K))

    def process_chunk(buf, nseg, xvec, ccol, clen, g, carry):
        def row_body(r, carry):
            thr8, pos8, zt0, zt1 = carry
            thr_f = _splat_f(jnp.max(jnp.where(io == r, thr8, NEG_INF)))
            pos = _splat_i(jnp.max(jnp.where(io == r, pos8, 0)))
            rbase = r * CAP

            def scan_block(lo, hi, unroll, thr_f, pos):
                @plsc.parallel_loop(lo, hi, unroll=unroll, carry=pos)
                def pos(i, pos):
                    v = buf[r, pl.ds(i * L, L)]
                    m = v > thr_f
                    pref = plsc.cumsum(m.astype(jnp.int32))
                    idx = rbase + (pos + pref - 1)
                    plsc.store_scatter(candbuf, [idx], v, mask=m)
                    return pos + plsc.all_reduce_population_count(m)

                pos_s = _scalar(pos)
                return lax.cond(pos_s > TRIGGER,
                                lambda: do_reselect(rbase, pos_s, MIDBITS),
                                lambda: (thr_f, pos))

            def seg_body(s, carry2):
                thr_f, pos = carry2
                return scan_block(s * SEGV, (s + 1) * SEGV, UNROLL,
                                  thr_f, pos)

            thr_f, pos = lax.fori_loop(0, nseg, seg_body, (thr_f, pos))
            if xvec:
                thr_f, pos = scan_block(nseg * SEGV, nseg * SEGV + xvec, 8,
                                        thr_f, pos)

            rl = g * 8 + r
            tg = tgtstage[pl.ds((rl // L) * L, L)]
            t_r = jnp.sum(jnp.where(io == (rl % L), tg, 0))
            o = t_r - ccol
            hit = (o >= 0) & (o < clen)
            oc = jnp.clip(o, 0, clen - 1)
            al = (oc // L) * L
            v16 = buf[r, pl.ds(al, L)]
            val = jnp.sum(jnp.where(io == (oc - al), v16, jnp.float32(0.0)))
            upd = hit.astype(jnp.float32) * val * \
                jnp.where(io == (rl % L), 1.0, 0.0)
            zt0 = zt0 + jnp.where(rl < L, upd, 0.0)
            zt1 = zt1 + jnp.where(rl >= L, upd, 0.0)

            thr8 = jnp.where(io == r, thr_f, thr8)
            pos8 = jnp.where(io == r, pos, pos8)
            return thr8, pos8, zt0, zt1

        return lax.fori_loop(0, 8, row_body, carry)

    def group_body(g, zt_carry):
        zt0, zt1 = zt_carry
        grow = base + g * 8
        nseg = CHUNK // (SEGV * L)

        def issue(c, buf, sem):
            pltpu.make_async_copy(
                input_hbm.at[pl.ds(grow, 8), pl.ds(c * CHUNK, CHUNK)],
                buf, sem).start()

        def wait(c, buf, sem):
            pltpu.make_async_copy(
                input_hbm.at[pl.ds(grow, 8), pl.ds(c * CHUNK, CHUNK)],
                buf, sem).wait()

        issue(0, databuf, semA)
        carry = (_splat_f(NEG_INF), _splat_i(0), zt0, zt1)

        def pair_body(i, ch_carry):
            cA = 2 * i
            wait(cA, databuf, semA)
            issue(cA + 1, databuf2, semB)
            ch_carry = process_chunk(databuf, nseg, 0, cA * CHUNK, CHUNK,
                                     g, ch_carry)
            wait(cA + 1, databuf2, semB)
            issue(cA + 2, databuf, semA)
            return process_chunk(databuf2, nseg, 0, (cA + 1) * CHUNK, CHUNK,
                                 g, ch_carry)

        carry = lax.fori_loop(0, NFULL // 2, pair_body, carry)

        wait(NFULL - 1, databuf, semA)
        pltpu.make_async_copy(
            input_hbm.at[pl.ds(grow, 8), pl.ds(NFULL * CHUNK, TAIL)],
            tailbuf, semB).start()
        carry = process_chunk(databuf, nseg, 0, (NFULL - 1) * CHUNK, CHUNK,
                              g, carry)
        pltpu.make_async_copy(
            input_hbm.at[pl.ds(grow, 8), pl.ds(NFULL * CHUNK, TAIL)],
            tailbuf, semB).wait()
        thr8, pos8, zt0, zt1 = process_chunk(
            tailbuf, TAIL_SEGS, TAIL_XVEC, NFULL * CHUNK, TAIL, g, carry)

        def fin_body(r, c):
            pos_s = jnp.max(jnp.where(io == r, pos8, 0))
            area = candbuf.at[pl.ds(r * CAP, CAP)]
            t100 = _find_kth_key(area, keybuf, pos_s, 31)
            tf = _inv_keymap_splat(t100)
            orow = outstage.at[r]
            for m in range(CW // L):
                orow[pl.ds(m * L, L)] = jnp.where(m * L + io < K, tf, NEG_INF)
            _compact_gt(area, orow, pos_s, tf)
            return c

        lax.fori_loop(0, 8, fin_body, 0)
        pltpu.sync_copy(outstage, cand_hbm.at[pl.ds(grow, 8), :])
        return zt0, zt1

    zt0, zt1 = lax.fori_loop(0, GROUPS, group_body,
                             (_splat_f(0.0), _splat_f(0.0)))
    ztstage[pl.ds(0, L)] = zt0
    ztstage[pl.ds(L, L)] = zt1
    pltpu.sync_copy(ztstage, zt_hbm.at[pl.ds(base, RPW)])


@functools.partial(jax.jit, static_argnums=())
def _sc_topk(input, target):
    mesh = plsc.VectorSubcoreMesh(core_axis_name="c", subcore_axis_name="s",
                                  num_cores=NC, num_subcores=NS)
    f = pl.kernel(
        _sc_body,
        out_type=(
            jax.ShapeDtypeStruct((N, CW), jnp.float32),
            jax.ShapeDtypeStruct((N,), jnp.float32),
        ),
        mesh=mesh,
        scratch_types=[
            pltpu.VMEM((8, CHUNK), jnp.float32),
            pltpu.VMEM((8, CHUNK), jnp.float32),
            pltpu.VMEM((8, TAIL), jnp.float32),
            pltpu.VMEM((8 * CAP,), jnp.float32),
            pltpu.VMEM((CAP,), jnp.int32),
            pltpu.VMEM((8, CW), jnp.float32),
            pltpu.VMEM((RPW,), jnp.int32),
            pltpu.VMEM((RPW,), jnp.float32),
            pltpu.SemaphoreType.DMA,
            pltpu.SemaphoreType.DMA,
        ],
        compiler_params=pltpu.CompilerParams(needs_layout_passes=False),
    )
    return f(input, target)


def _tail_body(cand_ref, extra_ref, zt_ref, tgt_ref, out_ref):
    v = cand_ref[...]
    lanes = lax.broadcasted_iota(jnp.int32, v.shape, 1)

    def rolled(x, s):
        left = jnp.concatenate([x[:, s:], x[:, :s]], axis=1)
        right = jnp.concatenate([x[:, -s:], x[:, :-s]], axis=1)
        return jnp.where((lanes & s) == 0, left, right)

    def bitonic(x, descending):
        k = 2
        while k <= CW:
            j = k // 2
            while j >= 1:
                p = rolled(x, j)
                take_max = ((lanes & k) == 0) == ((lanes & j) == 0)
                if not descending:
                    take_max = ~take_max
                x = jnp.where(take_max, jnp.maximum(x, p), jnp.minimum(x, p))
                j //= 2
            k *= 2
        return x

    ex = jnp.concatenate(
        [extra_ref[...], jnp.full((v.shape[0], CW - VX), NEG_INF, v.dtype)],
        axis=1)
    tl = tgt_ref[...] - V_SC
    ztfix = jnp.sum(jnp.where(lanes == tl, ex, 0.0), axis=1)

    va = bitonic(v, True)
    vb = bitonic(ex, False)
    vm = jnp.maximum(va, vb)
    j = CW // 2
    while j >= 1:
        p = rolled(vm, j)
        take_max = (lanes & j) == 0
        vm = jnp.where(take_max, jnp.maximum(vm, p), jnp.minimum(vm, p))
        j //= 2
    v = vm

    X = v * 0.5
    valid = lanes < K
    Xs = jnp.where(valid, X, 0.0)
    cum1 = Xs
    cum2 = Xs * Xs
    s = 1
    while s < CW:
        def shr(x, sh):
            return jnp.concatenate(
                [jnp.zeros((x.shape[0], sh), x.dtype), x[:, :-sh]], axis=1)
        cum1 = cum1 + shr(cum1, s)
        cum2 = cum2 + shr(cum2, s)
        s *= 2

    rho = (lanes + 1).astype(jnp.float32)
    mean = cum1 / rho
    meansq = cum2 / rho
    ss = rho * (meansq - mean * mean)
    delta = (1.0 - ss) / rho
    tau = mean - jnp.sqrt(jnp.clip(delta, 0.0, None))
    support_mask = (tau <= X) & valid
    support = jnp.sum(support_mask.astype(jnp.int32), axis=1, keepdims=True)
    tau_star = jnp.sum(jnp.where(lanes == support - 1, tau, 0.0), axis=1,
                       keepdims=True)
    p = jnp.where(valid, jnp.square(jnp.clip(X - tau_star, 0.0, None)), 0.0)
    p15 = p * jnp.sqrt(p)
    omega = (1.0 - jnp.sum(p15, axis=1)) / 0.75
    dot = jnp.sum(p * jnp.where(valid, v, 0.0), axis=1)
    loss = omega + dot - zt_ref[:, 0] - ztfix
    out_ref[...] = (jnp.sum(loss) / float(N)).reshape(1, 1)


def _tail(cand, extra, zt, target):
    out = pl.pallas_call(
        _tail_body,
        grid=(1,),
        in_specs=[
            pl.BlockSpec((N, CW), lambda i: (0, 0)),
            pl.BlockSpec((N, VX), lambda i: (0, 0)),
            pl.BlockSpec((N, 1), lambda i: (0, 0)),
            pl.BlockSpec((N, 1), lambda i: (0, 0)),
        ],
        out_specs=pl.BlockSpec((1, 1), lambda i: (0, 0)),
        out_shape=jax.ShapeDtypeStruct((1, 1), jnp.float32),
    )(cand, extra, zt.reshape(N, 1), target.reshape(N, 1))
    return out[0, 0]


def kernel(input, target):
    cand, zt = _sc_topk(input, target)
    extra = lax.slice(input, (0, V_SC), (N, V))
    return _tail(cand, extra, zt, target)
